# baseline (speedup 1.0000x reference)
import functools

import jax
import jax.numpy as jnp
from jax import lax
from jax.experimental import pallas as pl
from jax.experimental.pallas import tpu as pltpu
from jax.experimental.pallas import tpu_sc as plsc

H = 128
NL = 50000
NH = 25000
NC = 20000
EP = 160000
EN = 160000
ROUNDS = 3

NTILES = 16
CHUNK = 128

CPT = 160
GCHUNKS = CPT * NTILES
GCAP = GCHUNKS * CHUNK

NC_PAD = 20224
L2C_RANGE = NC_PAD // 2
L2C_SLAB = L2C_RANGE // NTILES
C2L_NR = 6
C2L_RANGE = 8448
C2L_SLAB = C2L_RANGE // NTILES
C2L_OUT_PAD = C2L_NR * C2L_RANGE
ACC_ROWS = L2C_RANGE + 8

BROW = 1000


def _sc_segsum_build():
    mesh = plsc.VectorSubcoreMesh(core_axis_name="c", subcore_axis_name="s")
    out_type = [
        jax.ShapeDtypeStruct((NC_PAD, H), jnp.float32),
        jax.ShapeDtypeStruct((C2L_OUT_PAD, H), jnp.float32),
    ]
    scratch = [
        pltpu.VMEM_SHARED((ACC_ROWS, H), jnp.float32),
        pltpu.VMEM((2, CHUNK), jnp.int32),
        pltpu.VMEM((2, CHUNK), jnp.int32),
        pltpu.VMEM((2, CHUNK, H), jnp.float32),
        pltpu.SemaphoreType.DMA,
    ]

    @functools.partial(pl.kernel, mesh=mesh, out_type=out_type,
                       scratch_types=scratch)
    def sc_segsum(lm, cm, l2c_src, l2c_dst, c2l_src, c2l_dst, zrows,
                  out_l2c, out_c2l,
                  acc, srcb, dstb, rowb, gsem):
        cid = lax.axis_index("c")
        sid = lax.axis_index("s")

        def phase(msg, srcg, dstg, g, slab, sent, out, out_base):
            pltpu.sync_copy(zrows.at[pl.ds(0, slab)],
                            acc.at[pl.ds(sid * slab, slab)])
            plsc.subcore_barrier()

            def load_idx(k, b):
                ch = k * NTILES + sid
                pltpu.sync_copy(srcg.at[g, ch], srcb.at[b])
                pltpu.sync_copy(dstg.at[g, ch], dstb.at[b])

            def start_gather(b):
                pltpu.async_copy(msg.at[srcb.at[b]], rowb.at[b], gsem)

            load_idx(0, 0)
            a0 = dstb[0, 0:16][0] < sent

            @pl.when(a0)
            def _():
                start_gather(0)

            def chunk_body(k, a_cur):
                b = lax.rem(k, 2)
                bn = lax.rem(k + 1, 2)

                @pl.when(k + 1 < CPT)
                def _():
                    load_idx(k + 1, bn)

                a_next = jnp.logical_and(k + 1 < CPT,
                                         dstb[bn, 0:16][0] < sent)

                @pl.when(a_next)
                def _():
                    start_gather(bn)

                @pl.when(a_cur)
                def _():
                    pltpu.make_async_copy(msg.at[srcb.at[b]], rowb.at[b],
                                          gsem).wait()
                    pltpu.sync_copy(rowb.at[b], acc.at[dstb.at[b]], add=True)

                return a_next

            lax.fori_loop(0, CPT, chunk_body, a0)
            plsc.subcore_barrier()
            pltpu.sync_copy(acc.at[pl.ds(sid * slab, slab)],
                            out.at[pl.ds(out_base + sid * slab, slab)])
            plsc.subcore_barrier()

        def corejob(l2c_g, c2l_gs):
            phase(lm, l2c_src, l2c_dst, l2c_g, L2C_SLAB, L2C_RANGE, out_l2c,
                  l2c_g * L2C_RANGE)
            for g in c2l_gs:
                phase(cm, c2l_src, c2l_dst, g, C2L_SLAB, C2L_RANGE, out_c2l,
                      g * C2L_RANGE)

        @pl.when(cid == 0)
        def _():
            corejob(0, (0, 2, 4))

        @pl.when(cid == 1)
        def _():
            corejob(1, (1, 3, 5))

    return sc_segsum


_sc_segsum = _sc_segsum_build()


def _dot(a, b):
    return jnp.dot(a, b, preferred_element_type=jnp.float32,
                   precision=lax.Precision.HIGHEST)


def _mlp_body(x_ref, w0, b0, w1, b1, w2, b2, out):
    x = x_ref[...]
    y = jnp.maximum(_dot(x, w0[...]) + b0[...], 0.0)
    y = jnp.maximum(_dot(y, w1[...]) + b1[...], 0.0)
    out[...] = _dot(y, w2[...]) + b2[...]


def _mlp_call(x, w0, b0, w1, b1, w2, b2):
    n = x.shape[0]
    full = lambda i: (0, 0)
    return pl.pallas_call(
        _mlp_body,
        grid=(n // BROW,),
        in_specs=[
            pl.BlockSpec((BROW, H), lambda i: (i, 0)),
            pl.BlockSpec((H, H), full), pl.BlockSpec((1, H), full),
            pl.BlockSpec((H, H), full), pl.BlockSpec((1, H), full),
            pl.BlockSpec((H, H), full), pl.BlockSpec((1, H), full),
        ],
        out_specs=pl.BlockSpec((BROW, H), lambda i: (i, 0)),
        out_shape=jax.ShapeDtypeStruct((n, H), jnp.float32),
    )(x, w0, b0, w1, b1, w2, b2)


def _gates(g, c):
    i = jax.nn.sigmoid(g[:, :H])
    f = jax.nn.sigmoid(g[:, H:2 * H])
    gg = jnp.tanh(g[:, 2 * H:3 * H])
    o = jax.nn.sigmoid(g[:, 3 * H:])
    c2 = f * c + i * gg
    return o * jnp.tanh(c2), c2


def _clstm_body(x_ref, h_ref, c_ref, wx, wh, b, h2o, c2o):
    g = _dot(x_ref[...], wx[...]) + _dot(h_ref[...], wh[...]) + b[...]
    h2, c2 = _gates(g, c_ref[...])
    h2o[...] = h2
    c2o[...] = c2


def _clstm_call(x, h, c, wx, wh, b):
    full = lambda i: (0, 0)
    row = lambda i: (i, 0)
    return pl.pallas_call(
        _clstm_body,
        grid=(NC // BROW,),
        in_specs=[
            pl.BlockSpec((BROW, H), row),
            pl.BlockSpec((BROW, H), row), pl.BlockSpec((BROW, H), row),
            pl.BlockSpec((H, 4 * H), full), pl.BlockSpec((H, 4 * H), full),
            pl.BlockSpec((1, 4 * H), full),
        ],
        out_specs=[pl.BlockSpec((BROW, H), row)] * 2,
        out_shape=[jax.ShapeDtypeStruct((NC, H), jnp.float32)] * 2,
    )(x, h, c, wx, wh, b)


def _llstm_body(x_ref, flip, h_ref, c_ref, wxa, wxb, wh, b, h2o, c2o):
    g = (_dot(x_ref[...], wxa[...]) + _dot(flip[...], wxb[...])
         + _dot(h_ref[...], wh[...]) + b[...])
    h2, c2 = _gates(g, c_ref[...])
    h2o[...] = h2
    c2o[...] = c2


def _llstm_call(x, lh, lc, wxa, wxb, wh, b):
    nblk = NL // BROW
    full = lambda i: (0, 0)
    row = lambda i: (i, 0)
    flip = lambda i: ((i + nblk // 2) % nblk, 0)
    return pl.pallas_call(
        _llstm_body,
        grid=(nblk,),
        in_specs=[
            pl.BlockSpec((BROW, H), row),
            pl.BlockSpec((BROW, H), flip),
            pl.BlockSpec((BROW, H), row), pl.BlockSpec((BROW, H), row),
            pl.BlockSpec((H, 4 * H), full), pl.BlockSpec((H, 4 * H), full),
            pl.BlockSpec((H, 4 * H), full), pl.BlockSpec((1, 4 * H), full),
        ],
        out_specs=[pl.BlockSpec((BROW, H), row)] * 2,
        out_shape=[jax.ShapeDtypeStruct((NL, H), jnp.float32)] * 2,
    )(x, lh, lh, lc, wxa, wxb, wh, b)


def _group_edges(src, dst, nranges, range_rows):
    gid = dst // range_rows
    onehot = (gid[:, None] ==
              jnp.arange(nranges, dtype=jnp.int32)[None, :]).astype(jnp.int32)
    rank = jnp.take_along_axis(jnp.cumsum(onehot, axis=0),
                               gid[:, None], axis=1)[:, 0] - 1
    dest = gid * GCAP + rank
    src_a = jnp.zeros((nranges * GCAP,), jnp.int32).at[dest].set(
        src, unique_indices=True, mode="promise_in_bounds")
    dst_a = jnp.full((nranges * GCAP,), range_rows, jnp.int32).at[dest].set(
        dst - gid * range_rows, unique_indices=True, mode="promise_in_bounds")
    return (src_a.reshape(nranges, GCHUNKS, CHUNK),
            dst_a.reshape(nranges, GCHUNKS, CHUNK))


def kernel(l_embedding, c_embedding, pos_edge_index, neg_edge_index,
           lW0, lb0, lW1, lb1, lW2, lb2,
           cW0, cb0, cW1, cb1, cW2, cb2,
           lu_Wih, lu_Whh, lu_bih, lu_bhh,
           cu_Wih, cu_Whh, cu_bih, cu_bhh):
    pos_src = pos_edge_index[0].astype(jnp.int32)
    pos_dst = pos_edge_index[1].astype(jnp.int32)
    neg_src = neg_edge_index[0].astype(jnp.int32)
    neg_dst = neg_edge_index[1].astype(jnp.int32)

    lit_adj = jnp.concatenate([pos_src, neg_src + NH])
    clause = jnp.concatenate([pos_dst, neg_dst])
    l2c_src, l2c_dst = _group_edges(lit_adj, clause, 2, L2C_RANGE)
    c2l_src, c2l_dst = _group_edges(clause, lit_adj, C2L_NR, C2L_RANGE)
    zrows = jnp.zeros((L2C_SLAB, H), jnp.float32)

    lw = [lW0.T, lb0.reshape(1, H), lW1.T, lb1.reshape(1, H),
          lW2.T, lb2.reshape(1, H)]
    cw = [cW0.T, cb0.reshape(1, H), cW1.T, cb1.reshape(1, H),
          cW2.T, cb2.reshape(1, H)]
    cu_wx = cu_Wih.T
    cu_wh = cu_Whh.T
    cu_b = (cu_bih + cu_bhh).reshape(1, 4 * H)
    lu_wxa = lu_Wih[:, :H].T
    lu_wxb = lu_Wih[:, H:].T
    lu_wh = lu_Whh.T
    lu_b = (lu_bih + lu_bhh).reshape(1, 4 * H)

    l_h = l_embedding
    l_c = jnp.zeros_like(l_embedding)
    c_h = c_embedding
    c_c = jnp.zeros_like(c_embedding)

    for _ in range(ROUNDS):
        lm = _mlp_call(l_h, *lw)
        cm = _mlp_call(c_h, *cw)
        l2c_msg, c2l_msg = _sc_segsum(lm, cm, l2c_src, l2c_dst,
                                      c2l_src, c2l_dst, zrows)
        new_ch, new_cc = _clstm_call(l2c_msg, c_h, c_c, cu_wx, cu_wh, cu_b)
        new_lh, new_lc = _llstm_call(c2l_msg, l_h, l_c,
                                     lu_wxa, lu_wxb, lu_wh, lu_b)
        c_h, c_c = new_ch, new_cc
        l_h, l_c = new_lh, new_lc

    return (l_h, c_h)


# speedup vs baseline: 1.8643x; 1.8643x over previous
import functools

import jax
import jax.numpy as jnp
from jax import lax
from jax.experimental import pallas as pl
from jax.experimental.pallas import tpu as pltpu
from jax.experimental.pallas import tpu_sc as plsc

H = 128
NL = 50000
NH = 25000
NC = 20000
EP = 160000
EN = 160000
ROUNDS = 3

NTILES = 16
CHUNK = 128

CPT = 160
GCHUNKS = CPT * NTILES
GCAP = GCHUNKS * CHUNK

NC_PAD = 20224
L2C_RANGE = NC_PAD // 2
L2C_SLAB = L2C_RANGE // NTILES
C2L_NR = 6
C2L_RANGE = 8448
C2L_SLAB = C2L_RANGE // NTILES
C2L_OUT_PAD = C2L_NR * C2L_RANGE
ACC_ROWS = L2C_RANGE + 8

BROW = 1000


def _sc_segsum_build():
    mesh = plsc.VectorSubcoreMesh(core_axis_name="c", subcore_axis_name="s")
    out_type = [
        jax.ShapeDtypeStruct((NC_PAD, H), jnp.float32),
        jax.ShapeDtypeStruct((C2L_OUT_PAD, H), jnp.float32),
    ]
    scratch = [
        pltpu.VMEM_SHARED((ACC_ROWS, H), jnp.float32),
        pltpu.VMEM((2, CHUNK), jnp.int32),
        pltpu.VMEM((2, CHUNK), jnp.int32),
        pltpu.VMEM((2, CHUNK, H), jnp.float32),
        pltpu.SemaphoreType.DMA,
    ]

    @functools.partial(pl.kernel, mesh=mesh, out_type=out_type,
                       scratch_types=scratch)
    def sc_segsum(lm, cm, l2c_src, l2c_dst, c2l_src, c2l_dst, zrows,
                  out_l2c, out_c2l,
                  acc, srcb, dstb, rowb, gsem):
        cid = lax.axis_index("c")
        sid = lax.axis_index("s")

        def phase(msg, srcg, dstg, g, slab, sent, out, out_base):
            pltpu.sync_copy(zrows.at[pl.ds(0, slab)],
                            acc.at[pl.ds(sid * slab, slab)])
            plsc.subcore_barrier()

            def load_idx(k, b):
                ch = k * NTILES + sid
                pltpu.sync_copy(srcg.at[g, ch], srcb.at[b])
                pltpu.sync_copy(dstg.at[g, ch], dstb.at[b])

            def start_gather(b):
                pltpu.async_copy(msg.at[srcb.at[b]], rowb.at[b], gsem)

            load_idx(0, 0)
            a0 = dstb[0, 0:16][0] < sent

            @pl.when(a0)
            def _():
                start_gather(0)

            def chunk_body(k, a_cur):
                b = lax.rem(k, 2)
                bn = lax.rem(k + 1, 2)

                @pl.when(k + 1 < CPT)
                def _():
                    load_idx(k + 1, bn)

                a_next = jnp.logical_and(k + 1 < CPT,
                                         dstb[bn, 0:16][0] < sent)

                @pl.when(a_next)
                def _():
                    start_gather(bn)

                @pl.when(a_cur)
                def _():
                    pltpu.make_async_copy(msg.at[srcb.at[b]], rowb.at[b],
                                          gsem).wait()
                    pltpu.sync_copy(rowb.at[b], acc.at[dstb.at[b]], add=True)

                return a_next

            lax.fori_loop(0, CPT, chunk_body, a0)
            plsc.subcore_barrier()
            pltpu.sync_copy(acc.at[pl.ds(sid * slab, slab)],
                            out.at[pl.ds(out_base + sid * slab, slab)])
            plsc.subcore_barrier()

        def corejob(l2c_g, c2l_gs):
            phase(lm, l2c_src, l2c_dst, l2c_g, L2C_SLAB, L2C_RANGE, out_l2c,
                  l2c_g * L2C_RANGE)
            for g in c2l_gs:
                phase(cm, c2l_src, c2l_dst, g, C2L_SLAB, C2L_RANGE, out_c2l,
                      g * C2L_RANGE)

        @pl.when(cid == 0)
        def _():
            corejob(0, (0, 2, 4))

        @pl.when(cid == 1)
        def _():
            corejob(1, (1, 3, 5))

    return sc_segsum


_sc_segsum = _sc_segsum_build()


def _dot(a, b):
    return jnp.dot(a, b, preferred_element_type=jnp.float32,
                   precision=lax.Precision.HIGHEST)


def _mlp_body(x_ref, w0, b0, w1, b1, w2, b2, out):
    x = x_ref[...]
    y = jnp.maximum(_dot(x, w0[...]) + b0[...], 0.0)
    y = jnp.maximum(_dot(y, w1[...]) + b1[...], 0.0)
    out[...] = _dot(y, w2[...]) + b2[...]


def _mlp_call(x, w0, b0, w1, b1, w2, b2):
    n = x.shape[0]
    full = lambda i: (0, 0)
    return pl.pallas_call(
        _mlp_body,
        grid=(n // BROW,),
        in_specs=[
            pl.BlockSpec((BROW, H), lambda i: (i, 0)),
            pl.BlockSpec((H, H), full), pl.BlockSpec((1, H), full),
            pl.BlockSpec((H, H), full), pl.BlockSpec((1, H), full),
            pl.BlockSpec((H, H), full), pl.BlockSpec((1, H), full),
        ],
        out_specs=pl.BlockSpec((BROW, H), lambda i: (i, 0)),
        out_shape=jax.ShapeDtypeStruct((n, H), jnp.float32),
    )(x, w0, b0, w1, b1, w2, b2)


def _gates(g, c):
    i = jax.nn.sigmoid(g[:, :H])
    f = jax.nn.sigmoid(g[:, H:2 * H])
    gg = jnp.tanh(g[:, 2 * H:3 * H])
    o = jax.nn.sigmoid(g[:, 3 * H:])
    c2 = f * c + i * gg
    return o * jnp.tanh(c2), c2


def _clstm_body(x_ref, h_ref, c_ref, wx, wh, b, h2o, c2o):
    g = _dot(x_ref[...], wx[...]) + _dot(h_ref[...], wh[...]) + b[...]
    h2, c2 = _gates(g, c_ref[...])
    h2o[...] = h2
    c2o[...] = c2


def _clstm_call(x, h, c, wx, wh, b):
    full = lambda i: (0, 0)
    row = lambda i: (i, 0)
    return pl.pallas_call(
        _clstm_body,
        grid=(NC // BROW,),
        in_specs=[
            pl.BlockSpec((BROW, H), row),
            pl.BlockSpec((BROW, H), row), pl.BlockSpec((BROW, H), row),
            pl.BlockSpec((H, 4 * H), full), pl.BlockSpec((H, 4 * H), full),
            pl.BlockSpec((1, 4 * H), full),
        ],
        out_specs=[pl.BlockSpec((BROW, H), row)] * 2,
        out_shape=[jax.ShapeDtypeStruct((NC, H), jnp.float32)] * 2,
    )(x, h, c, wx, wh, b)


def _llstm_body(x_ref, flip, h_ref, c_ref, wxa, wxb, wh, b, h2o, c2o):
    g = (_dot(x_ref[...], wxa[...]) + _dot(flip[...], wxb[...])
         + _dot(h_ref[...], wh[...]) + b[...])
    h2, c2 = _gates(g, c_ref[...])
    h2o[...] = h2
    c2o[...] = c2


def _llstm_call(x, lh, lc, wxa, wxb, wh, b):
    nblk = NL // BROW
    full = lambda i: (0, 0)
    row = lambda i: (i, 0)
    flip = lambda i: ((i + nblk // 2) % nblk, 0)
    return pl.pallas_call(
        _llstm_body,
        grid=(nblk,),
        in_specs=[
            pl.BlockSpec((BROW, H), row),
            pl.BlockSpec((BROW, H), flip),
            pl.BlockSpec((BROW, H), row), pl.BlockSpec((BROW, H), row),
            pl.BlockSpec((H, 4 * H), full), pl.BlockSpec((H, 4 * H), full),
            pl.BlockSpec((H, 4 * H), full), pl.BlockSpec((1, 4 * H), full),
        ],
        out_specs=[pl.BlockSpec((BROW, H), row)] * 2,
        out_shape=[jax.ShapeDtypeStruct((NL, H), jnp.float32)] * 2,
    )(x, lh, lh, lc, wxa, wxb, wh, b)


def _group_edges(src, dst, nranges, range_rows):
    e = src.shape[0]
    gid = dst // range_rows
    order = jnp.argsort(gid, stable=False)
    ssrc = jnp.concatenate([src[order], jnp.zeros((GCAP - e,), jnp.int32)])
    sdst = jnp.concatenate([dst[order], jnp.zeros((GCAP - e,), jnp.int32)])
    pos = jnp.arange(GCAP, dtype=jnp.int32)
    srcs, dsts = [], []
    start = jnp.int32(0)
    for g in range(nranges):
        cnt = jnp.sum((gid == g).astype(jnp.int32))
        valid = pos < cnt
        srcs.append(jnp.where(valid, jnp.roll(ssrc, -start), 0))
        dsts.append(jnp.where(valid, jnp.roll(sdst, -start) - g * range_rows,
                              range_rows))
        start = start + cnt
    return (jnp.stack(srcs).reshape(nranges, GCHUNKS, CHUNK),
            jnp.stack(dsts).reshape(nranges, GCHUNKS, CHUNK))


def kernel(l_embedding, c_embedding, pos_edge_index, neg_edge_index,
           lW0, lb0, lW1, lb1, lW2, lb2,
           cW0, cb0, cW1, cb1, cW2, cb2,
           lu_Wih, lu_Whh, lu_bih, lu_bhh,
           cu_Wih, cu_Whh, cu_bih, cu_bhh):
    pos_src = pos_edge_index[0].astype(jnp.int32)
    pos_dst = pos_edge_index[1].astype(jnp.int32)
    neg_src = neg_edge_index[0].astype(jnp.int32)
    neg_dst = neg_edge_index[1].astype(jnp.int32)

    lit_adj = jnp.concatenate([pos_src, neg_src + NH])
    clause = jnp.concatenate([pos_dst, neg_dst])
    l2c_src, l2c_dst = _group_edges(lit_adj, clause, 2, L2C_RANGE)
    c2l_src, c2l_dst = _group_edges(clause, lit_adj, C2L_NR, C2L_RANGE)
    zrows = jnp.zeros((L2C_SLAB, H), jnp.float32)

    lw = [lW0.T, lb0.reshape(1, H), lW1.T, lb1.reshape(1, H),
          lW2.T, lb2.reshape(1, H)]
    cw = [cW0.T, cb0.reshape(1, H), cW1.T, cb1.reshape(1, H),
          cW2.T, cb2.reshape(1, H)]
    cu_wx = cu_Wih.T
    cu_wh = cu_Whh.T
    cu_b = (cu_bih + cu_bhh).reshape(1, 4 * H)
    lu_wxa = lu_Wih[:, :H].T
    lu_wxb = lu_Wih[:, H:].T
    lu_wh = lu_Whh.T
    lu_b = (lu_bih + lu_bhh).reshape(1, 4 * H)

    l_h = l_embedding
    l_c = jnp.zeros_like(l_embedding)
    c_h = c_embedding
    c_c = jnp.zeros_like(c_embedding)

    for _ in range(ROUNDS):
        lm = _mlp_call(l_h, *lw)
        cm = _mlp_call(c_h, *cw)
        l2c_msg, c2l_msg = _sc_segsum(lm, cm, l2c_src, l2c_dst,
                                      c2l_src, c2l_dst, zrows)
        new_ch, new_cc = _clstm_call(l2c_msg, c_h, c_c, cu_wx, cu_wh, cu_b)
        new_lh, new_lc = _llstm_call(c2l_msg, l_h, l_c,
                                     lu_wxa, lu_wxb, lu_wh, lu_b)
        c_h, c_c = new_ch, new_cc
        l_h, l_c = new_lh, new_lc

    return (l_h, c_h)


# speedup vs baseline: 3.3283x; 1.7853x over previous
import functools

import jax
import jax.numpy as jnp
from jax import lax
from jax.experimental import pallas as pl
from jax.experimental.pallas import tpu as pltpu
from jax.experimental.pallas import tpu_sc as plsc

H = 128
NL = 50000
NH = 25000
NC = 20000
EP = 160000
EN = 160000
ROUNDS = 3

NTILES = 16
CHUNK = 128

CPT = 160
GCHUNKS = CPT * NTILES
GCAP = GCHUNKS * CHUNK

NC_PAD = 20224
L2C_RANGE = NC_PAD // 2
L2C_SLAB = L2C_RANGE // NTILES
C2L_NR = 6
C2L_RANGE = 8448
C2L_SLAB = C2L_RANGE // NTILES
C2L_OUT_PAD = C2L_NR * C2L_RANGE
ACC_ROWS = L2C_RANGE + 8

BROW = 1000


IB = 16
NSUPER = CPT // IB


def _phase(msg, srcg, dstg, g, slab, sent, out, out_base,
           sid, acc, srcb, dstb, rowb, gsem, zrows):
    pltpu.sync_copy(zrows.at[pl.ds(0, slab)],
                    acc.at[pl.ds(sid * slab, slab)])
    plsc.subcore_barrier()

    def load_batch(bat, b):
        pltpu.sync_copy(srcg.at[g, sid, pl.ds(bat * IB, IB)], srcb.at[b])
        pltpu.sync_copy(dstg.at[g, sid, pl.ds(bat * IB, IB)], dstb.at[b])

    load_batch(0, 0)
    a0 = dstb[0, 0, 0:16][0] < sent

    @pl.when(a0)
    def _():
        pltpu.async_copy(msg.at[srcb.at[0, 0]], rowb.at[0], gsem)

    def chunk_body(k, a_cur):
        b2 = lax.rem(k, 2)
        bn2 = lax.rem(k + 1, 2)
        bat = lax.div(k, IB)
        off = lax.rem(k, IB)
        bat_n = lax.div(k + 1, IB)
        off_n = lax.rem(k + 1, IB)
        ibb = lax.rem(bat, 2)
        ibn = lax.rem(bat_n, 2)

        @pl.when(jnp.logical_and(k + 1 < CPT, off_n == 0))
        def _():
            load_batch(bat_n, ibn)

        a_next = jnp.logical_and(k + 1 < CPT,
                                 dstb[ibn, off_n, 0:16][0] < sent)

        @pl.when(a_next)
        def _():
            pltpu.async_copy(msg.at[srcb.at[ibn, off_n]], rowb.at[bn2], gsem)

        @pl.when(a_cur)
        def _():
            pltpu.make_async_copy(msg.at[srcb.at[ibb, off]], rowb.at[b2],
                                  gsem).wait()
            pltpu.sync_copy(rowb.at[b2], acc.at[dstb.at[ibb, off]], add=True)

        return a_next

    lax.fori_loop(0, CPT, chunk_body, a0)
    plsc.subcore_barrier()
    pltpu.sync_copy(acc.at[pl.ds(sid * slab, slab)],
                    out.at[pl.ds(out_base + sid * slab, slab)])
    plsc.subcore_barrier()


def _sc_scratch():
    return [
        pltpu.VMEM_SHARED((ACC_ROWS, H), jnp.float32),
        pltpu.VMEM((2, IB, CHUNK), jnp.int32),
        pltpu.VMEM((2, IB, CHUNK), jnp.int32),
        pltpu.VMEM((2, CHUNK, H), jnp.float32),
        pltpu.SemaphoreType.DMA,
    ]


def _sc_l2c_build():
    mesh = plsc.VectorSubcoreMesh(core_axis_name="c", subcore_axis_name="s")

    @functools.partial(
        pl.kernel, mesh=mesh,
        out_type=jax.ShapeDtypeStruct((NC_PAD, H), jnp.float32),
        scratch_types=_sc_scratch())
    def sc_l2c(lm, src, dst, zrows, out, acc, srcb, dstb, rowb, gsem):
        cid = lax.axis_index("c")
        sid = lax.axis_index("s")
        args = (sid, acc, srcb, dstb, rowb, gsem, zrows)

        @pl.when(cid == 0)
        def _():
            _phase(lm, src, dst, 0, L2C_SLAB, L2C_RANGE, out, 0, *args)

        @pl.when(cid == 1)
        def _():
            _phase(lm, src, dst, 1, L2C_SLAB, L2C_RANGE, out, L2C_RANGE,
                   *args)

    return sc_l2c


def _sc_c2l_build():
    mesh = plsc.VectorSubcoreMesh(core_axis_name="c", subcore_axis_name="s")

    @functools.partial(
        pl.kernel, mesh=mesh,
        out_type=jax.ShapeDtypeStruct((C2L_OUT_PAD, H), jnp.float32),
        scratch_types=_sc_scratch())
    def sc_c2l(cm, src, dst, zrows, out, acc, srcb, dstb, rowb, gsem):
        cid = lax.axis_index("c")
        sid = lax.axis_index("s")
        args = (sid, acc, srcb, dstb, rowb, gsem, zrows)

        @pl.when(cid == 0)
        def _():
            for g in (0, 2, 4):
                _phase(cm, src, dst, g, C2L_SLAB, C2L_RANGE, out,
                       g * C2L_RANGE, *args)

        @pl.when(cid == 1)
        def _():
            for g in (1, 3, 5):
                _phase(cm, src, dst, g, C2L_SLAB, C2L_RANGE, out,
                       g * C2L_RANGE, *args)

    return sc_c2l


_sc_l2c = _sc_l2c_build()
_sc_c2l = _sc_c2l_build()


def _dot(a, b):
    return jnp.dot(a, b, preferred_element_type=jnp.float32,
                   precision=lax.Precision.HIGHEST)


def _mlp_body(x_ref, w0, b0, w1, b1, w2, b2, out):
    x = x_ref[...]
    y = jnp.maximum(_dot(x, w0[...]) + b0[...], 0.0)
    y = jnp.maximum(_dot(y, w1[...]) + b1[...], 0.0)
    out[...] = _dot(y, w2[...]) + b2[...]


def _mlp_call(x, w0, b0, w1, b1, w2, b2):
    n = x.shape[0]
    full = lambda i: (0, 0)
    return pl.pallas_call(
        _mlp_body,
        grid=(n // BROW,),
        in_specs=[
            pl.BlockSpec((BROW, H), lambda i: (i, 0)),
            pl.BlockSpec((H, H), full), pl.BlockSpec((1, H), full),
            pl.BlockSpec((H, H), full), pl.BlockSpec((1, H), full),
            pl.BlockSpec((H, H), full), pl.BlockSpec((1, H), full),
        ],
        out_specs=pl.BlockSpec((BROW, H), lambda i: (i, 0)),
        out_shape=jax.ShapeDtypeStruct((n, H), jnp.float32),
    )(x, w0, b0, w1, b1, w2, b2)


def _gates(g, c):
    i = jax.nn.sigmoid(g[:, :H])
    f = jax.nn.sigmoid(g[:, H:2 * H])
    gg = jnp.tanh(g[:, 2 * H:3 * H])
    o = jax.nn.sigmoid(g[:, 3 * H:])
    c2 = f * c + i * gg
    return o * jnp.tanh(c2), c2


def _clstm_body(x_ref, h_ref, c_ref, wx, wh, b, h2o, c2o):
    g = _dot(x_ref[...], wx[...]) + _dot(h_ref[...], wh[...]) + b[...]
    h2, c2 = _gates(g, c_ref[...])
    h2o[...] = h2
    c2o[...] = c2


def _clstm_call(x, h, c, wx, wh, b):
    full = lambda i: (0, 0)
    row = lambda i: (i, 0)
    return pl.pallas_call(
        _clstm_body,
        grid=(NC // BROW,),
        in_specs=[
            pl.BlockSpec((BROW, H), row),
            pl.BlockSpec((BROW, H), row), pl.BlockSpec((BROW, H), row),
            pl.BlockSpec((H, 4 * H), full), pl.BlockSpec((H, 4 * H), full),
            pl.BlockSpec((1, 4 * H), full),
        ],
        out_specs=[pl.BlockSpec((BROW, H), row)] * 2,
        out_shape=[jax.ShapeDtypeStruct((NC, H), jnp.float32)] * 2,
    )(x, h, c, wx, wh, b)


def _llstm_body(x_ref, flip, h_ref, c_ref, wxa, wxb, wh, b, h2o, c2o):
    g = (_dot(x_ref[...], wxa[...]) + _dot(flip[...], wxb[...])
         + _dot(h_ref[...], wh[...]) + b[...])
    h2, c2 = _gates(g, c_ref[...])
    h2o[...] = h2
    c2o[...] = c2


def _llstm_call(x, lh, lc, wxa, wxb, wh, b):
    nblk = NL // BROW
    full = lambda i: (0, 0)
    row = lambda i: (i, 0)
    flip = lambda i: ((i + nblk // 2) % nblk, 0)
    return pl.pallas_call(
        _llstm_body,
        grid=(nblk,),
        in_specs=[
            pl.BlockSpec((BROW, H), row),
            pl.BlockSpec((BROW, H), flip),
            pl.BlockSpec((BROW, H), row), pl.BlockSpec((BROW, H), row),
            pl.BlockSpec((H, 4 * H), full), pl.BlockSpec((H, 4 * H), full),
            pl.BlockSpec((H, 4 * H), full), pl.BlockSpec((1, 4 * H), full),
        ],
        out_specs=[pl.BlockSpec((BROW, H), row)] * 2,
        out_shape=[jax.ShapeDtypeStruct((NL, H), jnp.float32)] * 2,
    )(x, lh, lh, lc, wxa, wxb, wh, b)


def _group_edges(src, dst, nranges, range_rows):
    e = src.shape[0]
    gid = dst // range_rows
    order = jnp.argsort(gid, stable=False)
    ssrc = jnp.concatenate([src[order], jnp.zeros((GCAP - e,), jnp.int32)])
    sdst = jnp.concatenate([dst[order], jnp.zeros((GCAP - e,), jnp.int32)])
    pos = jnp.arange(GCAP, dtype=jnp.int32)
    srcs, dsts = [], []
    start = jnp.int32(0)
    for g in range(nranges):
        cnt = jnp.sum((gid == g).astype(jnp.int32))
        valid = pos < cnt
        srcs.append(jnp.where(valid, jnp.roll(ssrc, -start), 0))
        dsts.append(jnp.where(valid, jnp.roll(sdst, -start) - g * range_rows,
                              range_rows))
        start = start + cnt
    def lay(parts):
        a = jnp.stack(parts).reshape(nranges, CPT, NTILES, CHUNK)
        return a.transpose(0, 2, 1, 3)

    return lay(srcs), lay(dsts)


def kernel(l_embedding, c_embedding, pos_edge_index, neg_edge_index,
           lW0, lb0, lW1, lb1, lW2, lb2,
           cW0, cb0, cW1, cb1, cW2, cb2,
           lu_Wih, lu_Whh, lu_bih, lu_bhh,
           cu_Wih, cu_Whh, cu_bih, cu_bhh):
    pos_src = pos_edge_index[0].astype(jnp.int32)
    pos_dst = pos_edge_index[1].astype(jnp.int32)
    neg_src = neg_edge_index[0].astype(jnp.int32)
    neg_dst = neg_edge_index[1].astype(jnp.int32)

    lit_adj = jnp.concatenate([pos_src, neg_src + NH])
    clause = jnp.concatenate([pos_dst, neg_dst])
    l2c_src, l2c_dst = _group_edges(lit_adj, clause, 2, L2C_RANGE)
    c2l_src, c2l_dst = _group_edges(clause, lit_adj, C2L_NR, C2L_RANGE)
    zrows = jnp.zeros((L2C_SLAB, H), jnp.float32)

    lw = [lW0.T, lb0.reshape(1, H), lW1.T, lb1.reshape(1, H),
          lW2.T, lb2.reshape(1, H)]
    cw = [cW0.T, cb0.reshape(1, H), cW1.T, cb1.reshape(1, H),
          cW2.T, cb2.reshape(1, H)]
    cu_wx = cu_Wih.T
    cu_wh = cu_Whh.T
    cu_b = (cu_bih + cu_bhh).reshape(1, 4 * H)
    lu_wxa = lu_Wih[:, :H].T
    lu_wxb = lu_Wih[:, H:].T
    lu_wh = lu_Whh.T
    lu_b = (lu_bih + lu_bhh).reshape(1, 4 * H)

    l_h = l_embedding
    l_c = jnp.zeros_like(l_embedding)
    c_h = c_embedding
    c_c = jnp.zeros_like(c_embedding)

    for _ in range(ROUNDS):
        lm = _mlp_call(l_h, *lw)
        cm = _mlp_call(c_h, *cw)
        l2c_msg = _sc_l2c(lm, l2c_src, l2c_dst, zrows)
        c2l_msg = _sc_c2l(cm, c2l_src, c2l_dst, zrows)
        new_ch, new_cc = _clstm_call(l2c_msg, c_h, c_c, cu_wx, cu_wh, cu_b)
        new_lh, new_lc = _llstm_call(c2l_msg, l_h, l_c,
                                     lu_wxa, lu_wxb, lu_wh, lu_b)
        c_h, c_c = new_ch, new_cc
        l_h, l_c = new_lh, new_lc

    return (l_h, c_h)


# speedup vs baseline: 4.5223x; 1.3587x over previous
import functools

import jax
import jax.numpy as jnp
from jax import lax
from jax.experimental import pallas as pl
from jax.experimental.pallas import tpu as pltpu
from jax.experimental.pallas import tpu_sc as plsc

H = 128
NL = 50000
NH = 25000
NC = 20000
EP = 160000
EN = 160000
ROUNDS = 3

NTILES = 16
CHUNK = 128

CPT = 160
GCHUNKS = CPT * NTILES
GCAP = GCHUNKS * CHUNK

NC_PAD = 20224
L2C_RANGE = NC_PAD // 2
L2C_SLAB = L2C_RANGE // NTILES
C2L_NR = 6
C2L_RANGE = 8448
C2L_SLAB = C2L_RANGE // NTILES
C2L_OUT_PAD = C2L_NR * C2L_RANGE
ACC_ROWS = L2C_RANGE + 8

BROW = 1000


IB = 16
NSUPER = CPT // IB


def _phase(msg, srcg, dstg, g, slab, sent, out, out_base,
           sid, acc, srcb, dstb, rowb, gsem, zrows):
    pltpu.sync_copy(zrows.at[pl.ds(0, slab)],
                    acc.at[pl.ds(sid * slab, slab)])
    plsc.subcore_barrier()

    def load_batch(bat, b):
        pltpu.sync_copy(srcg.at[g, sid, pl.ds(bat * IB, IB)], srcb.at[b])
        pltpu.sync_copy(dstg.at[g, sid, pl.ds(bat * IB, IB)], dstb.at[b])

    load_batch(0, 0)
    a0 = dstb[0, 0, 0:16][0] < sent

    @pl.when(a0)
    def _():
        pltpu.async_copy(msg.at[srcb.at[0, 0]], rowb.at[0], gsem)

    def chunk_body(k, a_cur):
        b2 = lax.rem(k, 2)
        bn2 = lax.rem(k + 1, 2)
        bat = lax.div(k, IB)
        off = lax.rem(k, IB)
        bat_n = lax.div(k + 1, IB)
        off_n = lax.rem(k + 1, IB)
        ibb = lax.rem(bat, 2)
        ibn = lax.rem(bat_n, 2)

        @pl.when(jnp.logical_and(k + 1 < CPT, off_n == 0))
        def _():
            load_batch(bat_n, ibn)

        a_next = jnp.logical_and(k + 1 < CPT,
                                 dstb[ibn, off_n, 0:16][0] < sent)

        @pl.when(a_next)
        def _():
            pltpu.async_copy(msg.at[srcb.at[ibn, off_n]], rowb.at[bn2], gsem)

        @pl.when(a_cur)
        def _():
            pltpu.make_async_copy(msg.at[srcb.at[ibb, off]], rowb.at[b2],
                                  gsem).wait()
            pltpu.sync_copy(rowb.at[b2], acc.at[dstb.at[ibb, off]], add=True)

        return a_next

    lax.fori_loop(0, CPT, chunk_body, a0)
    plsc.subcore_barrier()
    pltpu.sync_copy(acc.at[pl.ds(sid * slab, slab)],
                    out.at[pl.ds(out_base + sid * slab, slab)])
    plsc.subcore_barrier()


def _sc_scratch():
    return [
        pltpu.VMEM_SHARED((ACC_ROWS, H), jnp.float32),
        pltpu.VMEM((2, IB, CHUNK), jnp.int32),
        pltpu.VMEM((2, IB, CHUNK), jnp.int32),
        pltpu.VMEM((2, CHUNK, H), jnp.float32),
        pltpu.SemaphoreType.DMA,
    ]


def _sc_l2c_build():
    mesh = plsc.VectorSubcoreMesh(core_axis_name="c", subcore_axis_name="s")

    @functools.partial(
        pl.kernel, mesh=mesh,
        out_type=jax.ShapeDtypeStruct((NC_PAD, H), jnp.float32),
        scratch_types=_sc_scratch())
    def sc_l2c(lm, src, dst, zrows, out, acc, srcb, dstb, rowb, gsem):
        cid = lax.axis_index("c")
        sid = lax.axis_index("s")
        args = (sid, acc, srcb, dstb, rowb, gsem, zrows)

        @pl.when(cid == 0)
        def _():
            _phase(lm, src, dst, 0, L2C_SLAB, L2C_RANGE, out, 0, *args)

        @pl.when(cid == 1)
        def _():
            _phase(lm, src, dst, 1, L2C_SLAB, L2C_RANGE, out, L2C_RANGE,
                   *args)

    return sc_l2c


def _sc_c2l_build():
    mesh = plsc.VectorSubcoreMesh(core_axis_name="c", subcore_axis_name="s")

    @functools.partial(
        pl.kernel, mesh=mesh,
        out_type=jax.ShapeDtypeStruct((C2L_OUT_PAD, H), jnp.float32),
        scratch_types=_sc_scratch())
    def sc_c2l(cm, src, dst, zrows, out, acc, srcb, dstb, rowb, gsem):
        cid = lax.axis_index("c")
        sid = lax.axis_index("s")
        args = (sid, acc, srcb, dstb, rowb, gsem, zrows)

        @pl.when(cid == 0)
        def _():
            for g in (0, 2, 4):
                _phase(cm, src, dst, g, C2L_SLAB, C2L_RANGE, out,
                       g * C2L_RANGE, *args)

        @pl.when(cid == 1)
        def _():
            for g in (1, 3, 5):
                _phase(cm, src, dst, g, C2L_SLAB, C2L_RANGE, out,
                       g * C2L_RANGE, *args)

    return sc_c2l


_sc_l2c = _sc_l2c_build()
_sc_c2l = _sc_c2l_build()


def _dot(a, b):
    ah = a.astype(jnp.bfloat16)
    al = (a - ah.astype(jnp.float32)).astype(jnp.bfloat16)
    bh = b.astype(jnp.bfloat16)
    bl = (b - bh.astype(jnp.float32)).astype(jnp.bfloat16)

    def d(u, v):
        return lax.dot_general(u, v, (((1,), (0,)), ((), ())),
                               preferred_element_type=jnp.float32)

    return d(ah, bh) + (d(ah, bl) + d(al, bh))


def _mlp_body(x_ref, w0, b0, w1, b1, w2, b2, out):
    x = x_ref[...]
    y = jnp.maximum(_dot(x, w0[...]) + b0[...], 0.0)
    y = jnp.maximum(_dot(y, w1[...]) + b1[...], 0.0)
    out[...] = _dot(y, w2[...]) + b2[...]


def _mlp_call(x, w0, b0, w1, b1, w2, b2):
    n = x.shape[0]
    full = lambda i: (0, 0)
    return pl.pallas_call(
        _mlp_body,
        grid=(n // BROW,),
        in_specs=[
            pl.BlockSpec((BROW, H), lambda i: (i, 0)),
            pl.BlockSpec((H, H), full), pl.BlockSpec((1, H), full),
            pl.BlockSpec((H, H), full), pl.BlockSpec((1, H), full),
            pl.BlockSpec((H, H), full), pl.BlockSpec((1, H), full),
        ],
        out_specs=pl.BlockSpec((BROW, H), lambda i: (i, 0)),
        out_shape=jax.ShapeDtypeStruct((n, H), jnp.float32),
    )(x, w0, b0, w1, b1, w2, b2)


def _gates(g, c):
    i = jax.nn.sigmoid(g[:, :H])
    f = jax.nn.sigmoid(g[:, H:2 * H])
    gg = jnp.tanh(g[:, 2 * H:3 * H])
    o = jax.nn.sigmoid(g[:, 3 * H:])
    c2 = f * c + i * gg
    return o * jnp.tanh(c2), c2


def _clstm_body(x_ref, h_ref, c_ref, wx, wh, b, h2o, c2o):
    g = _dot(x_ref[...], wx[...]) + _dot(h_ref[...], wh[...]) + b[...]
    h2, c2 = _gates(g, c_ref[...])
    h2o[...] = h2
    c2o[...] = c2


def _clstm_call(x, h, c, wx, wh, b):
    full = lambda i: (0, 0)
    row = lambda i: (i, 0)
    return pl.pallas_call(
        _clstm_body,
        grid=(NC // BROW,),
        in_specs=[
            pl.BlockSpec((BROW, H), row),
            pl.BlockSpec((BROW, H), row), pl.BlockSpec((BROW, H), row),
            pl.BlockSpec((H, 4 * H), full), pl.BlockSpec((H, 4 * H), full),
            pl.BlockSpec((1, 4 * H), full),
        ],
        out_specs=[pl.BlockSpec((BROW, H), row)] * 2,
        out_shape=[jax.ShapeDtypeStruct((NC, H), jnp.float32)] * 2,
    )(x, h, c, wx, wh, b)


def _llstm_body(x_ref, flip, h_ref, c_ref, wxa, wxb, wh, b, h2o, c2o):
    g = (_dot(x_ref[...], wxa[...]) + _dot(flip[...], wxb[...])
         + _dot(h_ref[...], wh[...]) + b[...])
    h2, c2 = _gates(g, c_ref[...])
    h2o[...] = h2
    c2o[...] = c2


def _llstm_call(x, lh, lc, wxa, wxb, wh, b):
    nblk = NL // BROW
    full = lambda i: (0, 0)
    row = lambda i: (i, 0)
    flip = lambda i: ((i + nblk // 2) % nblk, 0)
    return pl.pallas_call(
        _llstm_body,
        grid=(nblk,),
        in_specs=[
            pl.BlockSpec((BROW, H), row),
            pl.BlockSpec((BROW, H), flip),
            pl.BlockSpec((BROW, H), row), pl.BlockSpec((BROW, H), row),
            pl.BlockSpec((H, 4 * H), full), pl.BlockSpec((H, 4 * H), full),
            pl.BlockSpec((H, 4 * H), full), pl.BlockSpec((1, 4 * H), full),
        ],
        out_specs=[pl.BlockSpec((BROW, H), row)] * 2,
        out_shape=[jax.ShapeDtypeStruct((NL, H), jnp.float32)] * 2,
    )(x, lh, lh, lc, wxa, wxb, wh, b)


def _group_edges(src, dst, nranges, range_rows):
    e = src.shape[0]
    gid = dst // range_rows
    order = jnp.argsort(gid, stable=False)
    ssrc = jnp.concatenate([src[order], jnp.zeros((GCAP - e,), jnp.int32)])
    sdst = jnp.concatenate([dst[order], jnp.zeros((GCAP - e,), jnp.int32)])
    pos = jnp.arange(GCAP, dtype=jnp.int32)
    srcs, dsts = [], []
    start = jnp.int32(0)
    for g in range(nranges):
        cnt = jnp.sum((gid == g).astype(jnp.int32))
        valid = pos < cnt
        srcs.append(jnp.where(valid, jnp.roll(ssrc, -start), 0))
        dsts.append(jnp.where(valid, jnp.roll(sdst, -start) - g * range_rows,
                              range_rows))
        start = start + cnt
    def lay(parts):
        a = jnp.stack(parts).reshape(nranges, CPT, NTILES, CHUNK)
        return a.transpose(0, 2, 1, 3)

    return lay(srcs), lay(dsts)


def kernel(l_embedding, c_embedding, pos_edge_index, neg_edge_index,
           lW0, lb0, lW1, lb1, lW2, lb2,
           cW0, cb0, cW1, cb1, cW2, cb2,
           lu_Wih, lu_Whh, lu_bih, lu_bhh,
           cu_Wih, cu_Whh, cu_bih, cu_bhh):
    pos_src = pos_edge_index[0].astype(jnp.int32)
    pos_dst = pos_edge_index[1].astype(jnp.int32)
    neg_src = neg_edge_index[0].astype(jnp.int32)
    neg_dst = neg_edge_index[1].astype(jnp.int32)

    lit_adj = jnp.concatenate([pos_src, neg_src + NH])
    clause = jnp.concatenate([pos_dst, neg_dst])
    l2c_src, l2c_dst = _group_edges(lit_adj, clause, 2, L2C_RANGE)
    c2l_src, c2l_dst = _group_edges(clause, lit_adj, C2L_NR, C2L_RANGE)
    zrows = jnp.zeros((L2C_SLAB, H), jnp.float32)

    lw = [lW0.T, lb0.reshape(1, H), lW1.T, lb1.reshape(1, H),
          lW2.T, lb2.reshape(1, H)]
    cw = [cW0.T, cb0.reshape(1, H), cW1.T, cb1.reshape(1, H),
          cW2.T, cb2.reshape(1, H)]
    cu_wx = cu_Wih.T
    cu_wh = cu_Whh.T
    cu_b = (cu_bih + cu_bhh).reshape(1, 4 * H)
    lu_wxa = lu_Wih[:, :H].T
    lu_wxb = lu_Wih[:, H:].T
    lu_wh = lu_Whh.T
    lu_b = (lu_bih + lu_bhh).reshape(1, 4 * H)

    l_h = l_embedding
    l_c = jnp.zeros_like(l_embedding)
    c_h = c_embedding
    c_c = jnp.zeros_like(c_embedding)

    for _ in range(ROUNDS):
        lm = _mlp_call(l_h, *lw)
        cm = _mlp_call(c_h, *cw)
        l2c_msg = _sc_l2c(lm, l2c_src, l2c_dst, zrows)
        c2l_msg = _sc_c2l(cm, c2l_src, c2l_dst, zrows)
        new_ch, new_cc = _clstm_call(l2c_msg, c_h, c_c, cu_wx, cu_wh, cu_b)
        new_lh, new_lc = _llstm_call(c2l_msg, l_h, l_c,
                                     lu_wxa, lu_wxb, lu_wh, lu_b)
        c_h, c_c = new_ch, new_cc
        l_h, l_c = new_lh, new_lc

    return (l_h, c_h)


# speedup vs baseline: 5.2583x; 1.1627x over previous
import functools

import jax
import jax.numpy as jnp
from jax import lax
from jax.experimental import pallas as pl
from jax.experimental.pallas import tpu as pltpu
from jax.experimental.pallas import tpu_sc as plsc

H = 128
NL = 50000
NH = 25000
NC = 20000
EP = 160000
EN = 160000
ROUNDS = 3

NTILES = 16
CHUNK = 128

CPT = 160
GCHUNKS = CPT * NTILES
GCAP = GCHUNKS * CHUNK

NC_PAD = 20224
L2C_RANGE = NC_PAD // 2
L2C_SLAB = L2C_RANGE // NTILES
C2L_NR = 6
C2L_RANGE = 8448
C2L_SLAB = C2L_RANGE // NTILES
C2L_OUT_PAD = C2L_NR * C2L_RANGE
ACC_ROWS = L2C_RANGE + 8

BROW = 1000


IB = 16
NSUPER = CPT // IB


def _phase(msg, srcg, dstg, g, slab, sent, out, out_base,
           sid, acc, srcb, dstb, rowb, gsem, zrows):
    pltpu.sync_copy(zrows.at[pl.ds(0, slab)],
                    acc.at[pl.ds(sid * slab, slab)])
    plsc.subcore_barrier()

    def load_batch(bat, b):
        pltpu.sync_copy(srcg.at[g, sid, pl.ds(bat * IB, IB)], srcb.at[b])
        pltpu.sync_copy(dstg.at[g, sid, pl.ds(bat * IB, IB)], dstb.at[b])

    load_batch(0, 0)
    a0 = dstb[0, 0, 0:16][0] < sent

    @pl.when(a0)
    def _():
        pltpu.async_copy(msg.at[srcb.at[0, 0]], rowb.at[0], gsem)

    def chunk_body(k, a_cur):
        b2 = lax.rem(k, 2)
        bn2 = lax.rem(k + 1, 2)
        bat = lax.div(k, IB)
        off = lax.rem(k, IB)
        bat_n = lax.div(k + 1, IB)
        off_n = lax.rem(k + 1, IB)
        ibb = lax.rem(bat, 2)
        ibn = lax.rem(bat_n, 2)

        @pl.when(jnp.logical_and(k + 1 < CPT, off_n == 0))
        def _():
            load_batch(bat_n, ibn)

        a_next = jnp.logical_and(k + 1 < CPT,
                                 dstb[ibn, off_n, 0:16][0] < sent)

        @pl.when(a_next)
        def _():
            pltpu.async_copy(msg.at[srcb.at[ibn, off_n]], rowb.at[bn2], gsem)

        @pl.when(a_cur)
        def _():
            pltpu.make_async_copy(msg.at[srcb.at[ibb, off]], rowb.at[b2],
                                  gsem).wait()
            pltpu.sync_copy(rowb.at[b2], acc.at[dstb.at[ibb, off]], add=True)

        return a_next

    lax.fori_loop(0, CPT, chunk_body, a0)
    plsc.subcore_barrier()
    pltpu.sync_copy(acc.at[pl.ds(sid * slab, slab)],
                    out.at[pl.ds(out_base + sid * slab, slab)])
    plsc.subcore_barrier()


def _sc_scratch():
    return [
        pltpu.VMEM_SHARED((ACC_ROWS, H), jnp.float32),
        pltpu.VMEM((2, IB, CHUNK), jnp.int32),
        pltpu.VMEM((2, IB, CHUNK), jnp.int32),
        pltpu.VMEM((2, CHUNK, H), jnp.float32),
        pltpu.SemaphoreType.DMA,
    ]


def _sc_l2c_build():
    mesh = plsc.VectorSubcoreMesh(core_axis_name="c", subcore_axis_name="s")

    @functools.partial(
        pl.kernel, mesh=mesh,
        out_type=jax.ShapeDtypeStruct((NC_PAD, H), jnp.float32),
        scratch_types=_sc_scratch())
    def sc_l2c(lm, src, dst, zrows, out, acc, srcb, dstb, rowb, gsem):
        cid = lax.axis_index("c")
        sid = lax.axis_index("s")
        args = (sid, acc, srcb, dstb, rowb, gsem, zrows)

        @pl.when(cid == 0)
        def _():
            _phase(lm, src, dst, 0, L2C_SLAB, L2C_RANGE, out, 0, *args)

        @pl.when(cid == 1)
        def _():
            _phase(lm, src, dst, 1, L2C_SLAB, L2C_RANGE, out, L2C_RANGE,
                   *args)

    return sc_l2c


def _sc_c2l_build():
    mesh = plsc.VectorSubcoreMesh(core_axis_name="c", subcore_axis_name="s")

    @functools.partial(
        pl.kernel, mesh=mesh,
        out_type=jax.ShapeDtypeStruct((C2L_OUT_PAD, H), jnp.float32),
        scratch_types=_sc_scratch())
    def sc_c2l(cm, src, dst, zrows, out, acc, srcb, dstb, rowb, gsem):
        cid = lax.axis_index("c")
        sid = lax.axis_index("s")
        args = (sid, acc, srcb, dstb, rowb, gsem, zrows)

        @pl.when(cid == 0)
        def _():
            for g in (0, 2, 4):
                _phase(cm, src, dst, g, C2L_SLAB, C2L_RANGE, out,
                       g * C2L_RANGE, *args)

        @pl.when(cid == 1)
        def _():
            for g in (1, 3, 5):
                _phase(cm, src, dst, g, C2L_SLAB, C2L_RANGE, out,
                       g * C2L_RANGE, *args)

    return sc_c2l


_sc_l2c = _sc_l2c_build()
_sc_c2l = _sc_c2l_build()


def _dot(a, b):
    ah = a.astype(jnp.bfloat16)
    al = (a - ah.astype(jnp.float32)).astype(jnp.bfloat16)
    bh = b.astype(jnp.bfloat16)
    bl = (b - bh.astype(jnp.float32)).astype(jnp.bfloat16)

    def d(u, v):
        return lax.dot_general(u, v, (((1,), (0,)), ((), ())),
                               preferred_element_type=jnp.float32)

    return d(ah, bh) + (d(ah, bl) + d(al, bh))


def _mlp_body(x_ref, w0, b0, w1, b1, w2, b2, out):
    x = x_ref[...]
    y = jnp.maximum(_dot(x, w0[...]) + b0[...], 0.0)
    y = jnp.maximum(_dot(y, w1[...]) + b1[...], 0.0)
    out[...] = _dot(y, w2[...]) + b2[...]


def _mlp_call(x, w0, b0, w1, b1, w2, b2):
    n = x.shape[0]
    full = lambda i: (0, 0)
    return pl.pallas_call(
        _mlp_body,
        grid=(n // BROW,),
        in_specs=[
            pl.BlockSpec((BROW, H), lambda i: (i, 0)),
            pl.BlockSpec((H, H), full), pl.BlockSpec((1, H), full),
            pl.BlockSpec((H, H), full), pl.BlockSpec((1, H), full),
            pl.BlockSpec((H, H), full), pl.BlockSpec((1, H), full),
        ],
        out_specs=pl.BlockSpec((BROW, H), lambda i: (i, 0)),
        out_shape=jax.ShapeDtypeStruct((n, H), jnp.float32),
    )(x, w0, b0, w1, b1, w2, b2)


def _gates(g, c):
    i = jax.nn.sigmoid(g[:, :H])
    f = jax.nn.sigmoid(g[:, H:2 * H])
    gg = jnp.tanh(g[:, 2 * H:3 * H])
    o = jax.nn.sigmoid(g[:, 3 * H:])
    c2 = f * c + i * gg
    return o * jnp.tanh(c2), c2


def _clstm_body(x_ref, h_ref, c_ref, wx, wh, b, h2o, c2o):
    g = _dot(x_ref[...], wx[...]) + _dot(h_ref[...], wh[...]) + b[...]
    h2, c2 = _gates(g, c_ref[...])
    h2o[...] = h2
    c2o[...] = c2


def _clstm_call(x, h, c, wx, wh, b):
    full = lambda i: (0, 0)
    row = lambda i: (i, 0)
    return pl.pallas_call(
        _clstm_body,
        grid=(NC // BROW,),
        in_specs=[
            pl.BlockSpec((BROW, H), row),
            pl.BlockSpec((BROW, H), row), pl.BlockSpec((BROW, H), row),
            pl.BlockSpec((H, 4 * H), full), pl.BlockSpec((H, 4 * H), full),
            pl.BlockSpec((1, 4 * H), full),
        ],
        out_specs=[pl.BlockSpec((BROW, H), row)] * 2,
        out_shape=[jax.ShapeDtypeStruct((NC, H), jnp.float32)] * 2,
    )(x, h, c, wx, wh, b)


def _llstm_body(x_ref, flip, h_ref, c_ref, wxa, wxb, wh, b, h2o, c2o):
    g = (_dot(x_ref[...], wxa[...]) + _dot(flip[...], wxb[...])
         + _dot(h_ref[...], wh[...]) + b[...])
    h2, c2 = _gates(g, c_ref[...])
    h2o[...] = h2
    c2o[...] = c2


def _llstm_call(x, lh, lc, wxa, wxb, wh, b):
    nblk = NL // BROW
    full = lambda i: (0, 0)
    row = lambda i: (i, 0)
    flip = lambda i: ((i + nblk // 2) % nblk, 0)
    return pl.pallas_call(
        _llstm_body,
        grid=(nblk,),
        in_specs=[
            pl.BlockSpec((BROW, H), row),
            pl.BlockSpec((BROW, H), flip),
            pl.BlockSpec((BROW, H), row), pl.BlockSpec((BROW, H), row),
            pl.BlockSpec((H, 4 * H), full), pl.BlockSpec((H, 4 * H), full),
            pl.BlockSpec((H, 4 * H), full), pl.BlockSpec((1, 4 * H), full),
        ],
        out_specs=[pl.BlockSpec((BROW, H), row)] * 2,
        out_shape=[jax.ShapeDtypeStruct((NL, H), jnp.float32)] * 2,
    )(x, lh, lh, lc, wxa, wxb, wh, b)


def _lay(parts, nranges):
    a = jnp.stack(parts).reshape(nranges, CPT, NTILES, CHUNK)
    return a.transpose(0, 2, 1, 3)


def _build_groups(lit_adj, clause):
    e = lit_adj.shape[0]
    key = (lit_adj // C2L_RANGE) * 2 + clause // L2C_RANGE
    packed = lit_adj * 32768 + clause
    skey, sp = lax.sort((key, packed), num_keys=1, is_stable=False)
    starts = jnp.searchsorted(skey, jnp.arange(13, dtype=jnp.int32),
                              side="left").astype(jnp.int32)
    sp_pad = jnp.concatenate([sp, jnp.zeros((GCAP - e,), jnp.int32)])
    pos = jnp.arange(GCAP, dtype=jnp.int32)

    c2l_srcs, c2l_dsts = [], []
    for g in range(C2L_NR):
        s0 = starts[2 * g]
        ln = starts[2 * g + 2] - s0
        rolled = jnp.roll(sp_pad, -s0)
        valid = pos < ln
        c2l_srcs.append(jnp.where(valid, rolled % 32768, 0))
        c2l_dsts.append(jnp.where(valid, rolled // 32768 - g * C2L_RANGE,
                                  C2L_RANGE))

    l2c_srcs, l2c_dsts = [], []
    for g in range(2):
        src_g = jnp.zeros((GCAP,), jnp.int32)
        dst_g = jnp.full((GCAP,), L2C_RANGE, jnp.int32)
        off = jnp.int32(0)
        for j in range(C2L_NR):
            b = 2 * j + g
            ln = starts[b + 1] - starts[b]
            rolled = jnp.roll(sp_pad, off - starts[b])
            m = jnp.logical_and(pos >= off, pos < off + ln)
            src_g = jnp.where(m, rolled // 32768, src_g)
            dst_g = jnp.where(m, rolled % 32768 - g * L2C_RANGE, dst_g)
            off = off + ln
        l2c_srcs.append(src_g)
        l2c_dsts.append(dst_g)

    return (_lay(l2c_srcs, 2), _lay(l2c_dsts, 2),
            _lay(c2l_srcs, C2L_NR), _lay(c2l_dsts, C2L_NR))


def kernel(l_embedding, c_embedding, pos_edge_index, neg_edge_index,
           lW0, lb0, lW1, lb1, lW2, lb2,
           cW0, cb0, cW1, cb1, cW2, cb2,
           lu_Wih, lu_Whh, lu_bih, lu_bhh,
           cu_Wih, cu_Whh, cu_bih, cu_bhh):
    pos_src = pos_edge_index[0].astype(jnp.int32)
    pos_dst = pos_edge_index[1].astype(jnp.int32)
    neg_src = neg_edge_index[0].astype(jnp.int32)
    neg_dst = neg_edge_index[1].astype(jnp.int32)

    lit_adj = jnp.concatenate([pos_src, neg_src + NH])
    clause = jnp.concatenate([pos_dst, neg_dst])
    l2c_src, l2c_dst, c2l_src, c2l_dst = _build_groups(lit_adj, clause)
    zrows = jnp.zeros((L2C_SLAB, H), jnp.float32)

    lw = [lW0.T, lb0.reshape(1, H), lW1.T, lb1.reshape(1, H),
          lW2.T, lb2.reshape(1, H)]
    cw = [cW0.T, cb0.reshape(1, H), cW1.T, cb1.reshape(1, H),
          cW2.T, cb2.reshape(1, H)]
    cu_wx = cu_Wih.T
    cu_wh = cu_Whh.T
    cu_b = (cu_bih + cu_bhh).reshape(1, 4 * H)
    lu_wxa = lu_Wih[:, :H].T
    lu_wxb = lu_Wih[:, H:].T
    lu_wh = lu_Whh.T
    lu_b = (lu_bih + lu_bhh).reshape(1, 4 * H)

    l_h = l_embedding
    l_c = jnp.zeros_like(l_embedding)
    c_h = c_embedding
    c_c = jnp.zeros_like(c_embedding)

    for _ in range(ROUNDS):
        lm = _mlp_call(l_h, *lw)
        cm = _mlp_call(c_h, *cw)
        l2c_msg = _sc_l2c(lm, l2c_src, l2c_dst, zrows)
        c2l_msg = _sc_c2l(cm, c2l_src, c2l_dst, zrows)
        new_ch, new_cc = _clstm_call(l2c_msg, c_h, c_c, cu_wx, cu_wh, cu_b)
        new_lh, new_lc = _llstm_call(c2l_msg, l_h, l_c,
                                     lu_wxa, lu_wxb, lu_wh, lu_b)
        c_h, c_c = new_ch, new_cc
        l_h, l_c = new_lh, new_lc

    return (l_h, c_h)
